# baseline (speedup 1.0000x reference)
import functools

import jax
import jax.numpy as jnp
from jax import lax
from jax.experimental import pallas as pl
from jax.experimental.pallas import tpu as pltpu, tpu_sc as plsc

_PANEL = 128
_L = 16


def _make_panel_gather(V, D, B):
    info = plsc.get_sparse_core_info()
    NC, NS = info.num_cores, info.num_subcores
    NW = NC * NS
    n_full = V // _PANEL
    tail_start = n_full * _PANEL
    tail_n = V - tail_start
    tail_owner = n_full % NW
    mesh = plsc.VectorSubcoreMesh(core_axis_name="c", subcore_axis_name="s")

    @functools.partial(
        pl.kernel,
        mesh=mesh,
        out_type=jax.ShapeDtypeStruct((B, D), jnp.float32),
        compiler_params=pltpu.CompilerParams(needs_layout_passes=False),
        scratch_types=[
            pltpu.VMEM((B,), jnp.int32),
            pltpu.VMEM((B + _L,), jnp.int32),
            pltpu.VMEM((B + _L,), jnp.int32),
            pltpu.VMEM((B + _L,), jnp.int32),
            pltpu.VMEM((B + _L,), jnp.int32),
            pltpu.VMEM((D, _PANEL), jnp.float32),
            pltpu.VMEM((D, _PANEL), jnp.float32),
            pltpu.VMEM((tail_n, D), jnp.float32),
            pltpu.VMEM((_L, D), jnp.float32),
            pltpu.VMEM((_L, D), jnp.float32),
            pltpu.VMEM((_L, D), jnp.float32),
            pltpu.VMEM((_L, D), jnp.float32),
            pltpu.SemaphoreType.DMA,
            pltpu.SemaphoreType.DMA,
        ],
    )
    def gather_k(
        table_t_hbm, idx_hbm, tail_hbm, out_hbm,
        idx_all_v, myidx_v, mypos_v, subc_v, subj_v,
        panel_a, panel_b, tail_v, stage0_v, stage1_v, stage2_v, stage3_v,
        psem, osem,
    ):
        w = lax.axis_index("s") * NC + lax.axis_index("c")
        pltpu.sync_copy(idx_hbm, idx_all_v)
        pltpu.sync_copy(tail_hbm, tail_v)
        iota = lax.iota(jnp.int32, _L)
        dvecs = [iota + k * _L for k in range(D // _L)]

        def compress_append(ref_a, ref_b, base, xa, xb, m):
            mi = m.astype(jnp.int32)
            inc = plsc.cumsum(mi)
            pos = (inc - mi) + base
            plsc.store_scatter(ref_a, [pos], xa, mask=m)
            plsc.store_scatter(ref_b, [pos], xb, mask=m)
            return base + inc[_L - 1]

        @pl.loop(0, B // _L, init_carry=0, unroll=8)
        def cnt(k, cnt):
            v = idx_all_v[pl.ds(k * _L, _L)]
            mine = (lax.shift_right_logical(v, 7) & (NW - 1)) == w
            return compress_append(
                myidx_v, mypos_v, cnt, v, iota + k * _L, mine
            )

        n_trips = lax.div(cnt + (_L - 1), _L)
        n_slots = lax.div((n_full - 1) - w, NW) + 1

        def issue_panel(p, buf):
            col = pl.multiple_of(p * _PANEL, _PANEL)
            pltpu.async_copy(
                table_t_hbm.at[:, pl.ds(col, _PANEL)], buf, psem
            )

        def wait_panel():
            pltpu.make_async_copy(
                table_t_hbm.at[:, pl.ds(0, _PANEL)], panel_a, psem
            ).wait()

        def drain_block():
            pltpu.make_async_copy(
                stage0_v.at[pl.ds(0, _L)], out_hbm.at[pl.ds(0, _L)], osem
            ).wait()

        def build_sublist(p):
            @pl.loop(0, n_trips, init_carry=0)
            def scnt(t, scnt):
                v = myidx_v[pl.ds(t * _L, _L)]
                pos = mypos_v[pl.ds(t * _L, _L)]
                m = lax.shift_right_logical(v, 7) == p
                return compress_append(
                    subc_v, subj_v, scnt, v & (_PANEL - 1), pos, m
                )

            @pl.when(scnt > 0)
            def _():
                c16 = subc_v[pl.ds(0, _L)]
                j16 = subj_v[pl.ds(0, _L)]
                plsc.store_scatter(
                    subc_v, [scnt + iota], jnp.full((_L,), c16[0], jnp.int32)
                )
                plsc.store_scatter(
                    subj_v, [scnt + iota], jnp.full((_L,), j16[0], jnp.int32)
                )

            return scnt

        def extract_blocks(eblocks, cur):
            stages = (stage0_v, stage1_v, stage2_v, stage3_v)

            @pl.loop(0, eblocks)
            def _ex(e, /):
                cv = subc_v[pl.ds(e * _L, _L)]
                jv = subj_v[pl.ds(e * _L, _L)]

                def fill(stg):
                    for l in range(_L):
                        cvec = jnp.full((_L,), cv[l], jnp.int32)
                        for k in range(D // _L):
                            stg[l, pl.ds(k * _L, _L)] = (
                                plsc.load_gather(cur, [dvecs[k], cvec])
                            )
                        pltpu.async_copy(
                            stg.at[pl.ds(l, 1)],
                            out_hbm.at[pl.ds(jv[l], 1)],
                            osem,
                        )

                for q in range(4):
                    @pl.when((e & 3) == q)
                    def _(q=q):
                        fill(stages[q])

                @pl.when(e >= 3)
                def _():
                    drain_block()

            @pl.loop(0, jnp.minimum(eblocks, 3))
            def _left(e, /):
                drain_block()

        issue_panel(w, panel_a)

        @pl.loop(0, n_slots)
        def _slot(i, /):
            p = w + i * NW
            scnt = build_sublist(p)
            eblocks = lax.div(scnt + (_L - 1), _L)

            def process(cur, nxt):
                @pl.when(i + 1 < n_slots)
                def _():
                    issue_panel(p + NW, nxt)

                wait_panel()
                extract_blocks(eblocks, cur)

            @pl.when(i % 2 == 0)
            def _():
                process(panel_a, panel_b)

            @pl.when(i % 2 == 1)
            def _():
                process(panel_b, panel_a)

        @pl.when(w == tail_owner)
        def _():
            scnt = build_sublist(n_full)
            eblocks = lax.div(scnt + (_L - 1), _L)

            @pl.loop(0, eblocks)
            def _ex(e, /):
                cv = subc_v[pl.ds(e * _L, _L)]
                jv = subj_v[pl.ds(e * _L, _L)]
                for l in range(_L):
                    pltpu.async_copy(
                        tail_v.at[pl.ds(cv[l], 1)],
                        out_hbm.at[pl.ds(jv[l], 1)],
                        osem,
                    )

            @pl.loop(0, eblocks)
            def _tdrain(e, /):
                pltpu.make_async_copy(
                    tail_v.at[pl.ds(0, _L)],
                    out_hbm.at[pl.ds(0, _L)],
                    osem,
                ).wait()

    return gather_k


def kernel(data, channelindex):
    V, D = data.shape
    (B,) = channelindex.shape
    idx = channelindex.astype(jnp.int32)
    tail = data[(V // _PANEL) * _PANEL :, :]
    return _make_panel_gather(V, D, B)(data.T, idx, tail)


# speedup vs baseline: 1.5484x; 1.5484x over previous
import functools

import jax
import jax.numpy as jnp
from jax import lax
from jax.experimental import pallas as pl
from jax.experimental.pallas import tpu as pltpu, tpu_sc as plsc


def _make_gather(V, D, B):
    info = plsc.get_sparse_core_info()
    NC, NS = info.num_cores, info.num_subcores
    NW = NC * NS
    assert B % (8 * NW) == 0
    b_per_w = B // NW
    mesh = plsc.VectorSubcoreMesh(core_axis_name="c", subcore_axis_name="s")

    @functools.partial(
        pl.kernel,
        mesh=mesh,
        out_type=jax.ShapeDtypeStruct((B, D), jnp.float32),
        scratch_types=[
            pltpu.VMEM((b_per_w,), jnp.int32),
            pltpu.VMEM((b_per_w, D), jnp.float32),
            pltpu.SemaphoreType.DMA,
            pltpu.SemaphoreType.DMA,
        ],
    )
    def gather_k(table_hbm, idx_hbm, out_hbm, idx_v, rows_v, sem0, sem1):
        wid = lax.axis_index("s") * NC + lax.axis_index("c")
        base = wid * b_per_w
        pltpu.sync_copy(idx_hbm.at[pl.ds(base, b_per_w)], idx_v)
        sems = (sem0, sem1)

        def issue_chunk(c, sem):
            v = idx_v[pl.ds(c * 16, 16)]
            for l in range(16):
                pltpu.async_copy(
                    table_hbm.at[pl.ds(v[l], 1)],
                    rows_v.at[pl.ds(c * 16 + l, 1)],
                    sem,
                )

        def drain_chunk(sem):
            pltpu.make_async_copy(
                table_hbm.at[pl.ds(0, 16)], rows_v.at[pl.ds(0, 16)], sem
            ).wait()

        n_chunks = b_per_w // 16
        LAG = 4

        @pl.loop(0, n_chunks // 2)
        def _main(h):
            c = h * 2
            issue_chunk(c, sems[0])
            issue_chunk(c + 1, sems[1])

            @pl.when(c >= LAG)
            def _():
                drain_chunk(sems[0])
                drain_chunk(sems[1])

        for _ in range(LAG // 2):
            drain_chunk(sems[0])
            drain_chunk(sems[1])

        pltpu.sync_copy(rows_v, out_hbm.at[pl.ds(base, b_per_w)])

    return gather_k


def kernel(data, channelindex):
    V, D = data.shape
    (B,) = channelindex.shape
    idx = channelindex.astype(jnp.int32)
    return _make_gather(V, D, B)(data, idx)


# speedup vs baseline: 1.5882x; 1.0258x over previous
import functools

import jax
import jax.numpy as jnp
from jax import lax
from jax.experimental import pallas as pl
from jax.experimental.pallas import tpu as pltpu, tpu_sc as plsc


def _make_gather(V, D, B):
    info = plsc.get_sparse_core_info()
    NC, NS = info.num_cores, info.num_subcores
    NW = NC * NS
    assert B % (8 * NW) == 0
    b_per_w = B // NW
    mesh = plsc.VectorSubcoreMesh(core_axis_name="c", subcore_axis_name="s")

    @functools.partial(
        pl.kernel,
        mesh=mesh,
        out_type=jax.ShapeDtypeStruct((B, D), jnp.float32),
        scratch_types=[
            pltpu.VMEM((b_per_w,), jnp.int32),
            pltpu.VMEM((b_per_w, D), jnp.float32),
            pltpu.SemaphoreType.DMA,
            pltpu.SemaphoreType.DMA,
        ],
    )
    def gather_k(table_hbm, idx_hbm, out_hbm, idx_v, rows_v, sem0, sem1):
        wid = lax.axis_index("s") * NC + lax.axis_index("c")
        base = wid * b_per_w
        pltpu.sync_copy(idx_hbm.at[pl.ds(base, b_per_w)], idx_v)
        sems = (sem0, sem1)

        def issue_chunk(c, sem):
            v = idx_v[pl.ds(c * 16, 16)]
            for l in range(16):
                pltpu.async_copy(
                    table_hbm.at[pl.ds(v[l], 1)],
                    rows_v.at[pl.ds(c * 16 + l, 1)],
                    sem,
                )

        def drain_chunk(sem):
            pltpu.make_async_copy(
                table_hbm.at[pl.ds(0, 16)], rows_v.at[pl.ds(0, 16)], sem
            ).wait()

        n_chunks = b_per_w // 16
        LAG = 8

        @pl.loop(0, n_chunks // 2)
        def _main(h):
            c = h * 2
            issue_chunk(c, sems[0])
            issue_chunk(c + 1, sems[1])

            @pl.when(c >= LAG)
            def _():
                drain_chunk(sems[0])
                drain_chunk(sems[1])

        for _ in range(LAG // 2):
            drain_chunk(sems[0])
            drain_chunk(sems[1])

        pltpu.sync_copy(rows_v, out_hbm.at[pl.ds(base, b_per_w)])

    return gather_k


def kernel(data, channelindex):
    V, D = data.shape
    (B,) = channelindex.shape
    idx = channelindex.astype(jnp.int32)
    return _make_gather(V, D, B)(data, idx)


# speedup vs baseline: 1.5938x; 1.0035x over previous
import functools

import jax
import jax.numpy as jnp
from jax import lax
from jax.experimental import pallas as pl
from jax.experimental.pallas import tpu as pltpu, tpu_sc as plsc


def _make_gather(V, D, B):
    info = plsc.get_sparse_core_info()
    NC, NS = info.num_cores, info.num_subcores
    NW = NC * NS
    assert B % (8 * NW) == 0
    b_per_w = B // NW
    mesh = plsc.VectorSubcoreMesh(core_axis_name="c", subcore_axis_name="s")

    @functools.partial(
        pl.kernel,
        mesh=mesh,
        out_type=jax.ShapeDtypeStruct((B, D), jnp.float32),
        scratch_types=[
            pltpu.VMEM((b_per_w,), jnp.int32),
            pltpu.VMEM((b_per_w, D), jnp.float32),
            pltpu.SemaphoreType.DMA,
            pltpu.SemaphoreType.DMA,
        ],
    )
    def gather_k(table_hbm, idx_hbm, out_hbm, idx_v, rows_v, sem0, sem1):
        wid = lax.axis_index("s") * NC + lax.axis_index("c")
        base = wid * b_per_w
        pltpu.sync_copy(idx_hbm.at[pl.ds(base, b_per_w)], idx_v)
        sems = (sem0, sem1)

        def issue_chunk(c, sem):
            v = idx_v[pl.ds(c * 16, 16)]
            for l in range(16):
                pltpu.async_copy(
                    table_hbm.at[pl.ds(v[l], 1)],
                    rows_v.at[pl.ds(c * 16 + l, 1)],
                    sem,
                )

        def drain_chunk(sem):
            pltpu.make_async_copy(
                table_hbm.at[pl.ds(0, 16)], rows_v.at[pl.ds(0, 16)], sem
            ).wait()

        n_chunks = b_per_w // 16
        LAG = 16

        @pl.loop(0, n_chunks // 2)
        def _main(h):
            c = h * 2
            issue_chunk(c, sems[0])
            issue_chunk(c + 1, sems[1])

            @pl.when(c >= LAG)
            def _():
                drain_chunk(sems[0])
                drain_chunk(sems[1])

        for _ in range(LAG // 2):
            drain_chunk(sems[0])
            drain_chunk(sems[1])

        pltpu.sync_copy(rows_v, out_hbm.at[pl.ds(base, b_per_w)])

    return gather_k


def kernel(data, channelindex):
    V, D = data.shape
    (B,) = channelindex.shape
    idx = channelindex.astype(jnp.int32)
    return _make_gather(V, D, B)(data, idx)
